# baseline (speedup 1.0000x reference)
import jax
import jax.numpy as jnp
from jax import lax
from jax.experimental import pallas as pl
from jax.experimental.pallas import tpu as pltpu
from jax.experimental.pallas import tpu_sc as plsc

_BATCH = 4
_SEQ = 2048
_DIM = 1024
_NC = 2
_NS = 16
_NW = _NC * _NS
_ROWS_PER_W = _SEQ // _NW
_CH = 16
_NCHUNK = _ROWS_PER_W // _CH
_CHW = _CH * _DIM
_VEC = 16
_UNROLL = 8


def _row_copies(src_hbm, b, r0, buf, sem, to_hbm=False):
    copies = []
    for r in range(_CH):
        dst_sl = buf.at[pl.ds(r * _DIM, _DIM)]
        if b is None:
            hbm_sl = src_hbm.at[r0 + r, :]
        else:
            hbm_sl = src_hbm.at[b, r0 + r, :]
        if to_hbm:
            copies.append(pltpu.async_copy(dst_sl, hbm_sl, sem))
        else:
            copies.append(pltpu.async_copy(hbm_sl, dst_sl, sem))
    return copies


def _wait_all(copies):
    for cp in copies:
        cp.wait()


def _sc_body(in_hbm, tab_hbm, out_hbm,
             tbuf0, tbuf1, ibuf0, ibuf1, obuf0, obuf1,
             tsem0, tsem1, lsem0, lsem1, ssem0, ssem1):
    wid = lax.axis_index("s") * _NC + lax.axis_index("c")
    row0 = wid * _ROWS_PER_W

    tbufs = (tbuf0, tbuf1)
    ibufs = (ibuf0, ibuf1)
    obufs = (obuf0, obuf1)
    tsems = (tsem0, tsem1)
    lsems = (lsem0, lsem1)
    ssems = (ssem0, ssem1)

    tcopies = [None] * _NCHUNK
    tcopies[0] = _row_copies(tab_hbm, None, row0, tbufs[0], tsems[0])

    items = [(c, b) for c in range(_NCHUNK) for b in range(_BATCH)]
    n_items = len(items)
    lcopies = [None] * n_items
    scopies = [None] * n_items
    lcopies[0] = _row_copies(in_hbm, 0, row0, ibufs[0], lsems[0])
    lcopies[1] = _row_copies(in_hbm, 1, row0, ibufs[1], lsems[1])

    for k, (c, b) in enumerate(items):
        p = k % 2
        ibuf = ibufs[p]
        obuf = obufs[p]
        tbuf = tbufs[c % 2]

        if k % _BATCH == 0:
            _wait_all(tcopies[c])
            if c + 1 < _NCHUNK:
                tcopies[c + 1] = _row_copies(
                    tab_hbm, None, row0 + (c + 1) * _CH,
                    tbufs[(c + 1) % 2], tsems[(c + 1) % 2])

        if k >= 2:
            _wait_all(scopies[k - 2])
        _wait_all(lcopies[k])

        def add_body(i, _, ibuf=ibuf, tbuf=tbuf, obuf=obuf):
            for u in range(_UNROLL):
                sl = pl.ds((i * _UNROLL + u) * _VEC, _VEC)
                obuf[sl] = ibuf[sl] + tbuf[sl]
            return 0

        lax.fori_loop(0, _CHW // (_VEC * _UNROLL), add_body, 0,
                      unroll=False)

        nk = k + 2
        if nk < n_items:
            ncb = items[nk]
            lcopies[nk] = _row_copies(
                in_hbm, ncb[1], row0 + ncb[0] * _CH, ibufs[p], lsems[p])

        scopies[k] = _row_copies(
            out_hbm, b, row0 + c * _CH, obuf, ssems[p], to_hbm=True)

    _wait_all(scopies[n_items - 2])
    _wait_all(scopies[n_items - 1])


def kernel(inputs, pos_table):
    batch, seq_len, out_dim = inputs.shape
    mesh = plsc.VectorSubcoreMesh(core_axis_name="c", subcore_axis_name="s")
    sc = pl.kernel(
        _sc_body,
        mesh=mesh,
        out_type=jax.ShapeDtypeStruct((batch, seq_len, out_dim),
                                      jnp.float32),
        scratch_types=[
            pltpu.VMEM((_CHW,), jnp.float32),
            pltpu.VMEM((_CHW,), jnp.float32),
            pltpu.VMEM((_CHW,), jnp.float32),
            pltpu.VMEM((_CHW,), jnp.float32),
            pltpu.VMEM((_CHW,), jnp.float32),
            pltpu.VMEM((_CHW,), jnp.float32),
            pltpu.SemaphoreType.DMA,
            pltpu.SemaphoreType.DMA,
            pltpu.SemaphoreType.DMA,
            pltpu.SemaphoreType.DMA,
            pltpu.SemaphoreType.DMA,
            pltpu.SemaphoreType.DMA,
        ],
    )
    return sc(inputs, pos_table)


# speedup vs baseline: 2.3574x; 2.3574x over previous
import jax
import jax.numpy as jnp
from jax.experimental import pallas as pl

_SEQ_BLOCK = 512


def _add_kernel(in_ref, table_ref, out_ref):
    out_ref[...] = in_ref[...] + table_ref[...][None, :, :]


def kernel(inputs, pos_table):
    batch, seq_len, out_dim = inputs.shape
    grid = (seq_len // _SEQ_BLOCK,)
    return pl.pallas_call(
        _add_kernel,
        grid=grid,
        in_specs=[
            pl.BlockSpec((batch, _SEQ_BLOCK, out_dim), lambda i: (0, i, 0)),
            pl.BlockSpec((_SEQ_BLOCK, out_dim), lambda i: (i, 0)),
        ],
        out_specs=pl.BlockSpec((batch, _SEQ_BLOCK, out_dim), lambda i: (0, i, 0)),
        out_shape=jax.ShapeDtypeStruct(inputs.shape, inputs.dtype),
    )(inputs, pos_table)
